# baseline (speedup 1.0000x reference)
import jax
import jax.numpy as jnp
from jax import lax
from jax.experimental import pallas as pl
from jax.experimental.pallas import tpu as pltpu
from jax.experimental.pallas import tpu_sc as plsc

_NUM_CORES = 2
_NUM_SUBCORES = 16
_NUM_WORKERS = _NUM_CORES * _NUM_SUBCORES

_GROUP = 128
_LANE = 16
_NBUF = 4
_NTBUF = 2


def _gather_call(hist, dim, vocab, idx_dtype):
    sub = dim // 8

    mesh = plsc.VectorSubcoreMesh(core_axis_name="c", subcore_axis_name="s")

    def body(idx_hbm, table_hbm, out_hbm, idx_v, bufs, tbufs, *sems):
        gsems = sems[:_NBUF]
        wsems = sems[_NBUF:]
        wid = lax.axis_index("s") * _NUM_CORES + lax.axis_index("c")
        pltpu.sync_copy(idx_hbm.at[:, pl.ds(wid * _GROUP, _GROUP)], idx_v)

        def start_gather(b, h):
            pltpu.async_copy(
                table_hbm.at[idx_v.at[h]], bufs.at[b], gsems[b]
            )

        def wait_gather(b):
            pltpu.make_async_copy(
                table_hbm.at[pl.ds(0, _GROUP)], bufs.at[b], gsems[b]
            ).wait()

        def transpose(b, t):
            def tr_body(tr, c):
                for s in range(8):
                    d = tr * 8 + s
                    cols = jnp.full((_LANE,), d, jnp.int32)
                    for j in range(_GROUP // _LANE):
                        rows = lax.iota(jnp.int32, _LANE) + (j * _LANE)
                        v = plsc.load_gather(bufs.at[b], [rows, cols])
                        tbufs[t, tr, s, pl.ds(j * _LANE, _LANE)] = v
                return c

            lax.fori_loop(0, dim // 8, tr_body, 0)

        def start_write(t, h):
            pltpu.async_copy(
                tbufs.at[t], out_hbm.at[h, :, wid], wsems[t]
            )

        def wait_write(t):
            pltpu.make_async_copy(
                tbufs.at[t], out_hbm.at[0, :, 0], wsems[t]
            ).wait()

        for b in range(_NBUF):
            start_gather(b, b)

        def step(o, c):
            for b in range(_NBUF):
                h = o * _NBUF + b
                t = b % _NTBUF
                wait_gather(b)

                @pl.when(h >= _NTBUF)
                def _():
                    wait_write(t)

                transpose(b, t)

                @pl.when(h + _NBUF < hist)
                def _():
                    start_gather(b, h + _NBUF)

                start_write(t, h)
            return c

        lax.fori_loop(0, hist // _NBUF, step, 0)
        for t in range(_NTBUF):
            wait_write(t)

    return pl.kernel(
        body,
        out_type=jax.ShapeDtypeStruct(
            (hist, 8, _NUM_WORKERS, sub, _GROUP), jnp.float32
        ),
        mesh=mesh,
        scratch_types=[
            pltpu.VMEM((hist, _GROUP), idx_dtype),
            pltpu.VMEM((_NBUF, _GROUP, dim), jnp.float32),
            pltpu.VMEM((_NTBUF, 8, sub, _GROUP), jnp.float32),
        ]
        + [pltpu.SemaphoreType.DMA] * (_NBUF + _NTBUF),
        compiler_params=pltpu.CompilerParams(
            use_tc_tiling_on_sc=False, needs_layout_passes=False
        ),
    )


def kernel(inputs, matrix):
    batch, hist = inputs.shape
    vocab, dim = matrix.shape
    assert batch == _GROUP * _NUM_WORKERS
    idx_t = inputs.T
    out5 = _gather_call(hist, dim, vocab, idx_t.dtype)(idx_t, matrix)
    return jnp.transpose(out5, (2, 4, 0, 1, 3)).reshape(batch, hist, dim)


# speedup vs baseline: 1.1767x; 1.1767x over previous
import jax
import jax.numpy as jnp
from jax import lax
from jax.experimental import pallas as pl
from jax.experimental.pallas import tpu as pltpu
from jax.experimental.pallas import tpu_sc as plsc

_NUM_CORES = 2
_NUM_SUBCORES = 16
_NUM_WORKERS = _NUM_CORES * _NUM_SUBCORES

_GROUP = 128
_LANE = 16
_NBUF = 4
_NTBUF = 2


def _gather_call(hist, dim, vocab, idx_dtype):
    sub = dim // 8

    mesh = plsc.VectorSubcoreMesh(core_axis_name="c", subcore_axis_name="s")

    def body(idx_hbm, table_hbm, out_hbm, idx_v, bufs, tbufs, *sems):
        gsems = sems[:_NBUF]
        wsems = sems[_NBUF:]
        wid = lax.axis_index("s") * _NUM_CORES + lax.axis_index("c")
        pltpu.sync_copy(idx_hbm.at[:, pl.ds(wid * _GROUP, _GROUP)], idx_v)

        def start_gather(b, h):
            pltpu.async_copy(
                table_hbm.at[idx_v.at[h]], bufs.at[b], gsems[b]
            )

        def wait_gather(b):
            pltpu.make_async_copy(
                table_hbm.at[pl.ds(0, _GROUP)], bufs.at[b], gsems[b]
            ).wait()

        def transpose(b, t):
            def tr_body(tr, c):
                for s2 in range(0, 8, 2):
                    vals = []
                    for s in (s2, s2 + 1):
                        d = tr * 8 + s
                        cols = jnp.full((_LANE,), d, jnp.int32)
                        for j in range(_GROUP // _LANE):
                            rows = lax.iota(jnp.int32, _LANE) + (j * _LANE)
                            vals.append(
                                (s, j, plsc.load_gather(bufs.at[b], [rows, cols]))
                            )
                    for s, j, v in vals:
                        tbufs[t, tr, s, pl.ds(j * _LANE, _LANE)] = v
                return c

            lax.fori_loop(0, dim // 8, tr_body, 0)

        def start_write(t, h):
            pltpu.async_copy(
                tbufs.at[t], out_hbm.at[h, :, wid], wsems[t]
            )

        def wait_write(t):
            pltpu.make_async_copy(
                tbufs.at[t], out_hbm.at[0, :, 0], wsems[t]
            ).wait()

        for b in range(_NBUF):
            start_gather(b, b)

        def step(o, c):
            for b in range(_NBUF):
                h = o * _NBUF + b
                t = b % _NTBUF
                wait_gather(b)

                @pl.when(h >= _NTBUF)
                def _():
                    wait_write(t)

                transpose(b, t)

                @pl.when(h + _NBUF < hist)
                def _():
                    start_gather(b, h + _NBUF)

                start_write(t, h)
            return c

        lax.fori_loop(0, hist // _NBUF, step, 0)
        for t in range(_NTBUF):
            wait_write(t)

    return pl.kernel(
        body,
        out_type=jax.ShapeDtypeStruct(
            (hist, 8, _NUM_WORKERS, sub, _GROUP), jnp.float32
        ),
        mesh=mesh,
        scratch_types=[
            pltpu.VMEM((hist, _GROUP), idx_dtype),
            pltpu.VMEM((_NBUF, _GROUP, dim), jnp.float32),
            pltpu.VMEM((_NTBUF, 8, sub, _GROUP), jnp.float32),
        ]
        + [pltpu.SemaphoreType.DMA] * (_NBUF + _NTBUF),
        compiler_params=pltpu.CompilerParams(
            use_tc_tiling_on_sc=False, needs_layout_passes=False
        ),
    )


def kernel(inputs, matrix):
    batch, hist = inputs.shape
    vocab, dim = matrix.shape
    assert batch == _GROUP * _NUM_WORKERS
    idx_t = inputs.T
    out5 = _gather_call(hist, dim, vocab, idx_t.dtype)(idx_t, matrix)
    return jnp.transpose(out5, (2, 4, 0, 1, 3)).reshape(batch, hist, dim)


# speedup vs baseline: 2.3420x; 1.9903x over previous
import jax
import jax.numpy as jnp
from jax import lax
from jax.experimental import pallas as pl
from jax.experimental.pallas import tpu as pltpu
from jax.experimental.pallas import tpu_sc as plsc

_NUM_CORES = 2
_NUM_SUBCORES = 16
_NUM_WORKERS = _NUM_CORES * _NUM_SUBCORES

_GROUP = 128
_LANE = 16
_NBUF = 4
_NTBUF = 2


def _gather_call(hist, dim, vocab, idx_dtype):
    sub = dim // 8

    mesh = plsc.VectorSubcoreMesh(core_axis_name="c", subcore_axis_name="s")

    def body(idx_hbm, table_hbm, out_hbm, idx_v, bufs, tbufs, *sems):
        gsems = sems[:_NBUF]
        wsems = sems[_NBUF:]
        wid = lax.axis_index("s") * _NUM_CORES + lax.axis_index("c")
        pltpu.sync_copy(idx_hbm.at[:, pl.ds(wid * _GROUP, _GROUP)], idx_v)

        def start_gather(b, h):
            pltpu.async_copy(
                table_hbm.at[idx_v.at[h]], bufs.at[b], gsems[b]
            )

        def wait_gather(b):
            pltpu.make_async_copy(
                table_hbm.at[pl.ds(0, _GROUP)], bufs.at[b], gsems[b]
            ).wait()

        def transpose(b, t):
            lanes = lax.iota(jnp.int32, _LANE)

            def dd_body(dd, c):
                for jj in range(_GROUP // _LANE):
                    rows = lanes + jj * _LANE
                    vals = []
                    for r in range(_LANE):
                        cols = dd * _LANE + ((lanes + r) & (_LANE - 1))
                        v = plsc.load_gather(bufs.at[b], [rows, cols])
                        vals.append((cols, v))
                    for cols, v in vals:
                        plsc.store_scatter(
                            tbufs.at[t], [cols >> 3, cols & 7, rows], v
                        )
                return c

            lax.fori_loop(0, dim // _LANE, dd_body, 0)

        def start_write(t, h):
            pltpu.async_copy(
                tbufs.at[t], out_hbm.at[h, :, wid], wsems[t]
            )

        def wait_write(t):
            pltpu.make_async_copy(
                tbufs.at[t], out_hbm.at[0, :, 0], wsems[t]
            ).wait()

        for b in range(_NBUF):
            start_gather(b, b)

        def step(o, c):
            for b in range(_NBUF):
                h = o * _NBUF + b
                t = b % _NTBUF
                wait_gather(b)

                @pl.when(h >= _NTBUF)
                def _():
                    wait_write(t)

                transpose(b, t)

                @pl.when(h + _NBUF < hist)
                def _():
                    start_gather(b, h + _NBUF)

                start_write(t, h)
            return c

        lax.fori_loop(0, hist // _NBUF, step, 0)
        for t in range(_NTBUF):
            wait_write(t)

    return pl.kernel(
        body,
        out_type=jax.ShapeDtypeStruct(
            (hist, 8, _NUM_WORKERS, sub, _GROUP), jnp.float32
        ),
        mesh=mesh,
        scratch_types=[
            pltpu.VMEM((hist, _GROUP), idx_dtype),
            pltpu.VMEM((_NBUF, _GROUP, dim), jnp.float32),
            pltpu.VMEM((_NTBUF, 8, sub, _GROUP), jnp.float32),
        ]
        + [pltpu.SemaphoreType.DMA] * (_NBUF + _NTBUF),
        compiler_params=pltpu.CompilerParams(
            use_tc_tiling_on_sc=False, needs_layout_passes=False
        ),
    )


def kernel(inputs, matrix):
    batch, hist = inputs.shape
    vocab, dim = matrix.shape
    assert batch == _GROUP * _NUM_WORKERS
    idx_t = inputs.T
    out5 = _gather_call(hist, dim, vocab, idx_t.dtype)(idx_t, matrix)
    return jnp.transpose(out5, (2, 4, 0, 1, 3)).reshape(batch, hist, dim)


# speedup vs baseline: 2.8998x; 1.2382x over previous
import jax
import jax.numpy as jnp
from jax import lax
from jax.experimental import pallas as pl
from jax.experimental.pallas import tpu as pltpu
from jax.experimental.pallas import tpu_sc as plsc

_NUM_CORES = 2
_NUM_SUBCORES = 16
_NUM_WORKERS = _NUM_CORES * _NUM_SUBCORES

_GROUP = 128
_LANE = 16
_NBUF = 2
_NTBUF = 2

_MESH = plsc.VectorSubcoreMesh(core_axis_name="c", subcore_axis_name="s")


def _transpose_call(vocab, dim):
    n_cols = vocab // _GROUP
    n_tail = vocab - n_cols * _GROUP
    base, extra = divmod(n_cols, _NUM_WORKERS)

    def body(mat_hbm, tail_hbm, out_hbm, s_in, s_out, tmp, *sems):
        isems = sems[:2]
        osems = sems[2:]
        wid = lax.axis_index("s") * _NUM_CORES + lax.axis_index("c")
        nq = base + jnp.where(wid < extra, 1, 0)
        lanes = lax.iota(jnp.int32, _LANE)

        def start_in(p, q):
            tc = wid + q * _NUM_WORKERS
            pltpu.async_copy(
                mat_hbm.at[:, pl.ds(tc * _GROUP, _GROUP)], s_in.at[p], isems[p]
            )

        def wait_in(p):
            pltpu.make_async_copy(
                mat_hbm.at[:, pl.ds(0, _GROUP)], s_in.at[p], isems[p]
            ).wait()

        def start_out(p, q):
            tc = wid + q * _NUM_WORKERS
            pltpu.async_copy(
                s_out.at[p], out_hbm.at[pl.ds(tc * (_GROUP // 2), _GROUP // 2)],
                osems[p],
            )

        def wait_out(p):
            pltpu.make_async_copy(
                s_out.at[p], out_hbm.at[pl.ds(0, _GROUP // 2)], osems[p]
            ).wait()

        def transpose(p):
            def m_body(m, c):
                lv = lanes + m * _LANE
                vpv = lv >> 1
                cbase = (lv & 1) << 6
                for dd in range(dim // _LANE):
                    vals = []
                    for r in range(_LANE):
                        dv = dd * _LANE + ((lanes + r) & (_LANE - 1))
                        v = plsc.load_gather(s_in.at[p], [dv, lv])
                        vals.append((dv, v))
                    for dv, v in vals:
                        plsc.store_scatter(
                            s_out.at[p], [vpv, cbase | dv], v
                        )
                return c

            lax.fori_loop(0, _GROUP // _LANE, m_body, 0)

        def step(q, c):
            p = lax.rem(q, 2)

            def phase(p):
                wait_in(p)

                @pl.when(q >= 2)
                def _():
                    wait_out(p)

                transpose(p)

                @pl.when(q + 2 < nq)
                def _():
                    start_in(p, q + 2)

                start_out(p, q)

            @pl.when(p == 0)
            def _():
                phase(0)

            @pl.when(p == 1)
            def _():
                phase(1)

            return c

        start_in(0, 0)
        start_in(1, 1)
        lax.fori_loop(0, nq, step, 0)
        wait_out(0)
        wait_out(1)

        if n_tail:
            @pl.when(wid == 0)
            def _():
                pltpu.sync_copy(tail_hbm, tmp)
                pltpu.sync_copy(
                    tmp,
                    out_hbm.at[pl.ds(n_cols * (_GROUP // 2), n_tail // 2)],
                )

    return pl.kernel(
        body,
        out_type=jax.ShapeDtypeStruct((vocab // 2, 2 * dim), jnp.float32),
        mesh=_MESH,
        scratch_types=[
            pltpu.VMEM((2, dim, _GROUP), jnp.float32),
            pltpu.VMEM((2, _GROUP // 2, 2 * dim), jnp.float32),
            pltpu.VMEM((max(n_tail, 2) // 2, 2 * dim), jnp.float32),
        ]
        + [pltpu.SemaphoreType.DMA] * 4,
        compiler_params=pltpu.CompilerParams(
            use_tc_tiling_on_sc=True, needs_layout_passes=False
        ),
    )


def _gather_call(hist, dim, idx_dtype):
    pdim = 2 * dim

    def body(idx_hbm, table_hbm, out_hbm, idx_v, idx2, bufs, tbufs, *sems):
        gsems = sems[:_NBUF]
        wsems = sems[_NBUF:]
        wid = lax.axis_index("s") * _NUM_CORES + lax.axis_index("c")
        lanes = lax.iota(jnp.int32, _LANE)
        pltpu.sync_copy(idx_hbm.at[:, pl.ds(wid * _GROUP, _GROUP)], idx_v)

        def start_gather(b, h):
            for k in range(_GROUP // _LANE):
                idx2[b, pl.ds(k * _LANE, _LANE)] = (
                    idx_v[h, pl.ds(k * _LANE, _LANE)] >> 1
                )
            pltpu.async_copy(table_hbm.at[idx2.at[b]], bufs.at[b], gsems[b])

        def wait_gather(b):
            pltpu.make_async_copy(
                table_hbm.at[pl.ds(0, _GROUP)], bufs.at[b], gsems[b]
            ).wait()

        def transpose(b, t, h):
            def jj_body(jj, c):
                rows = lanes + jj * _LANE
                par = (idx_v[h, pl.ds(jj * _LANE, _LANE)] & 1) << 6
                for dd in range(dim // _LANE):
                    vals = []
                    for r in range(_LANE):
                        dloc = dd * _LANE + ((lanes + r) & (_LANE - 1))
                        v = plsc.load_gather(bufs.at[b], [rows, dloc + par])
                        vals.append((dloc, v))
                    for dloc, v in vals:
                        plsc.store_scatter(
                            tbufs.at[t], [dloc >> 3, dloc & 7, rows], v
                        )
                return c

            lax.fori_loop(0, _GROUP // _LANE, jj_body, 0)

        def start_write(t, h):
            pltpu.async_copy(tbufs.at[t], out_hbm.at[h, :, wid], wsems[t])

        def wait_write(t):
            pltpu.make_async_copy(
                tbufs.at[t], out_hbm.at[0, :, 0], wsems[t]
            ).wait()

        for b in range(_NBUF):
            start_gather(b, b)

        def step(o, c):
            for b in range(_NBUF):
                h = o * _NBUF + b
                t = b % _NTBUF
                wait_gather(b)

                @pl.when(h >= _NTBUF)
                def _():
                    wait_write(t)

                transpose(b, t, h)

                @pl.when(h + _NBUF < hist)
                def _():
                    start_gather(b, h + _NBUF)

                start_write(t, h)
            return c

        lax.fori_loop(0, hist // _NBUF, step, 0)
        for t in range(_NTBUF):
            wait_write(t)

    return pl.kernel(
        body,
        out_type=jax.ShapeDtypeStruct(
            (hist, 8, _NUM_WORKERS, dim // 8, _GROUP), jnp.float32
        ),
        mesh=_MESH,
        scratch_types=[
            pltpu.VMEM((hist, _GROUP), idx_dtype),
            pltpu.VMEM((_NBUF, _GROUP), jnp.int32),
            pltpu.VMEM((_NBUF, _GROUP, pdim), jnp.float32),
            pltpu.VMEM((_NTBUF, 8, dim // 8, _GROUP), jnp.float32),
        ]
        + [pltpu.SemaphoreType.DMA] * (_NBUF + _NTBUF),
        compiler_params=pltpu.CompilerParams(
            use_tc_tiling_on_sc=False, needs_layout_passes=False
        ),
    )


def kernel(inputs, matrix):
    batch, hist = inputs.shape
    vocab, dim = matrix.shape
    assert batch == _GROUP * _NUM_WORKERS
    n_tail = vocab % _GROUP
    tail = matrix[vocab - n_tail :].reshape(max(n_tail, 2) // 2, 2 * dim)
    packed = _transpose_call(vocab, dim)(matrix.T, tail)
    idx_t = inputs.T
    out5 = _gather_call(hist, dim, idx_t.dtype)(idx_t, packed)
    return jnp.transpose(out5, (2, 4, 0, 1, 3)).reshape(batch, hist, dim)


# speedup vs baseline: 3.5534x; 1.2254x over previous
import jax
import jax.numpy as jnp
from jax import lax
from jax.experimental import pallas as pl
from jax.experimental.pallas import tpu as pltpu
from jax.experimental.pallas import tpu_sc as plsc

_NUM_CORES = 2
_NUM_SUBCORES = 16
_NUM_WORKERS = _NUM_CORES * _NUM_SUBCORES

_GROUP = 128
_LANE = 16
_NBUF = 4
_NTBUF = 2

_MESH = plsc.VectorSubcoreMesh(core_axis_name="c", subcore_axis_name="s")


def _transpose_call(vocab, dim):
    n_cols = vocab // _GROUP
    n_tail = vocab - n_cols * _GROUP
    base, extra = divmod(n_cols, _NUM_WORKERS)

    def body(mat_hbm, tail_hbm, out_hbm, s_in, s_out, tmp, *sems):
        isems = sems[:2]
        osems = sems[2:]
        wid = lax.axis_index("s") * _NUM_CORES + lax.axis_index("c")
        nq = base + jnp.where(wid < extra, 1, 0)
        lanes = lax.iota(jnp.int32, _LANE)

        def start_in(p, q):
            tc = wid + q * _NUM_WORKERS
            pltpu.async_copy(
                mat_hbm.at[:, pl.ds(tc * _GROUP, _GROUP)], s_in.at[p], isems[p]
            )

        def wait_in(p):
            pltpu.make_async_copy(
                mat_hbm.at[:, pl.ds(0, _GROUP)], s_in.at[p], isems[p]
            ).wait()

        def start_out(p, q):
            tc = wid + q * _NUM_WORKERS
            pltpu.async_copy(
                s_out.at[p], out_hbm.at[pl.ds(tc * (_GROUP // 2), _GROUP // 2)],
                osems[p],
            )

        def wait_out(p):
            pltpu.make_async_copy(
                s_out.at[p], out_hbm.at[pl.ds(0, _GROUP // 2)], osems[p]
            ).wait()

        def transpose(p):
            def m_body(m, c):
                lv = lanes + m * _LANE
                vpv = lv >> 1
                cbase = (lv & 1) << 6
                for dd in range(dim // _LANE):
                    vals = []
                    for r in range(_LANE):
                        dv = dd * _LANE + ((lanes + r) & (_LANE - 1))
                        v = plsc.load_gather(s_in.at[p], [dv, lv])
                        vals.append((dv, v))
                    for dv, v in vals:
                        plsc.store_scatter(
                            s_out.at[p], [vpv, cbase | dv], v
                        )
                return c

            lax.fori_loop(0, _GROUP // _LANE, m_body, 0)

        def step(q, c):
            p = lax.rem(q, 2)

            def phase(p):
                wait_in(p)

                @pl.when(q >= 2)
                def _():
                    wait_out(p)

                transpose(p)

                @pl.when(q + 2 < nq)
                def _():
                    start_in(p, q + 2)

                start_out(p, q)

            @pl.when(p == 0)
            def _():
                phase(0)

            @pl.when(p == 1)
            def _():
                phase(1)

            return c

        start_in(0, 0)
        start_in(1, 1)
        lax.fori_loop(0, nq, step, 0)
        wait_out(0)
        wait_out(1)

        if n_tail:
            @pl.when(wid == 0)
            def _():
                pltpu.sync_copy(tail_hbm, tmp)
                pltpu.sync_copy(
                    tmp,
                    out_hbm.at[pl.ds(n_cols * (_GROUP // 2), n_tail // 2)],
                )

    return pl.kernel(
        body,
        out_type=jax.ShapeDtypeStruct((vocab // 2, 2 * dim), jnp.float32),
        mesh=_MESH,
        scratch_types=[
            pltpu.VMEM((2, dim, _GROUP), jnp.float32),
            pltpu.VMEM((2, _GROUP // 2, 2 * dim), jnp.float32),
            pltpu.VMEM((max(n_tail, 2) // 2, 2 * dim), jnp.float32),
        ]
        + [pltpu.SemaphoreType.DMA] * 4,
        compiler_params=pltpu.CompilerParams(
            use_tc_tiling_on_sc=True, needs_layout_passes=False
        ),
    )


def _gather_call(hist, dim, idx_dtype):
    def body(idx_hbm, table_hbm, out_hbm, idx_v, bufs, tbufs, *sems):
        gsems = sems[:_NBUF]
        wsems = sems[_NBUF:]
        wid = lax.axis_index("s") * _NUM_CORES + lax.axis_index("c")
        lanes = lax.iota(jnp.int32, _LANE)
        pltpu.sync_copy(idx_hbm.at[:, pl.ds(wid * _GROUP, _GROUP)], idx_v)

        def start_gather(b, h):
            pltpu.async_copy(table_hbm.at[idx_v.at[h]], bufs.at[b], gsems[b])

        def wait_gather(b):
            pltpu.make_async_copy(
                table_hbm.at[pl.ds(0, _GROUP)], bufs.at[b], gsems[b]
            ).wait()

        def transpose(b, t, h):
            def jj_body(jj, c):
                rows = lanes + jj * _LANE
                for dd in range(dim // _LANE):
                    vals = []
                    for r in range(_LANE):
                        dloc = dd * _LANE + ((lanes + r) & (_LANE - 1))
                        v = plsc.load_gather(bufs.at[b], [rows, dloc])
                        vals.append((dloc, v))
                    for dloc, v in vals:
                        plsc.store_scatter(
                            tbufs.at[t], [dloc >> 3, dloc & 7, rows], v
                        )
                return c

            lax.fori_loop(0, _GROUP // _LANE, jj_body, 0)

        def start_write(t, h):
            pltpu.async_copy(tbufs.at[t], out_hbm.at[h, :, wid], wsems[t])

        def wait_write(t):
            pltpu.make_async_copy(
                tbufs.at[t], out_hbm.at[0, :, 0], wsems[t]
            ).wait()

        for b in range(_NBUF):
            start_gather(b, b)

        def step(o, c):
            for b in range(_NBUF):
                h = o * _NBUF + b
                t = b % _NTBUF
                wait_gather(b)

                @pl.when(h >= _NTBUF)
                def _():
                    wait_write(t)

                transpose(b, t, h)

                @pl.when(h + _NBUF < hist)
                def _():
                    start_gather(b, h + _NBUF)

                start_write(t, h)
            return c

        lax.fori_loop(0, hist // _NBUF, step, 0)
        for t in range(_NTBUF):
            wait_write(t)

    return pl.kernel(
        body,
        out_type=jax.ShapeDtypeStruct(
            (hist, 8, _NUM_WORKERS, dim // 8, _GROUP), jnp.float32
        ),
        mesh=_MESH,
        scratch_types=[
            pltpu.VMEM((hist, _GROUP), idx_dtype),
            pltpu.VMEM((_NBUF, _GROUP, dim), jnp.float32),
            pltpu.VMEM((_NTBUF, 8, dim // 8, _GROUP), jnp.float32),
        ]
        + [pltpu.SemaphoreType.DMA] * (_NBUF + _NTBUF),
        compiler_params=pltpu.CompilerParams(
            use_tc_tiling_on_sc=False, needs_layout_passes=False
        ),
    )


def kernel(inputs, matrix):
    batch, hist = inputs.shape
    vocab, dim = matrix.shape
    assert batch == _GROUP * _NUM_WORKERS
    n_tail = vocab % _GROUP
    tail = matrix[vocab - n_tail :].reshape(max(n_tail, 2) // 2, 2 * dim)
    packed = _transpose_call(vocab, dim)(matrix.T, tail)
    table_rm = packed.reshape(vocab, dim)
    idx_t = inputs.T
    out5 = _gather_call(hist, dim, idx_t.dtype)(idx_t, table_rm)
    return jnp.transpose(out5, (2, 4, 0, 1, 3)).reshape(batch, hist, dim)


# speedup vs baseline: 3.8929x; 1.0955x over previous
import jax
import jax.numpy as jnp
from jax import lax
from jax.experimental import pallas as pl
from jax.experimental.pallas import tpu as pltpu
from jax.experimental.pallas import tpu_sc as plsc

_NUM_CORES = 2
_NUM_SUBCORES = 16
_NUM_WORKERS = _NUM_CORES * _NUM_SUBCORES

_GROUP = 128
_LANE = 16
_NBUF = 4
_NTBUF = 2

_MESH = plsc.VectorSubcoreMesh(core_axis_name="c", subcore_axis_name="s")


def _transpose_call(vocab, dim):
    n_cols = vocab // _GROUP
    n_tail = vocab - n_cols * _GROUP
    base, extra = divmod(n_cols, _NUM_WORKERS)

    nph = 3

    def body(mat_hbm, tail_hbm, out_hbm, s_in, s_out, tmp, *sems):
        isems = sems[:nph]
        osems = sems[nph:]
        wid = lax.axis_index("s") * _NUM_CORES + lax.axis_index("c")
        nq = base + jnp.where(wid < extra, 1, 0)
        lanes = lax.iota(jnp.int32, _LANE)

        def start_in(p, q):
            tc = wid + q * _NUM_WORKERS
            pltpu.async_copy(
                mat_hbm.at[:, pl.ds(tc * _GROUP, _GROUP)], s_in.at[p], isems[p]
            )

        def wait_in(p):
            pltpu.make_async_copy(
                mat_hbm.at[:, pl.ds(0, _GROUP)], s_in.at[p], isems[p]
            ).wait()

        def start_out(p, q):
            tc = wid + q * _NUM_WORKERS
            pltpu.async_copy(
                s_out.at[p], out_hbm.at[pl.ds(tc * (_GROUP // 2), _GROUP // 2)],
                osems[p],
            )

        def wait_out(p):
            pltpu.make_async_copy(
                s_out.at[p], out_hbm.at[pl.ds(0, _GROUP // 2)], osems[p]
            ).wait()

        def transpose(p):
            def m_body(m, c):
                lv = lanes + m * _LANE
                vpv = lv >> 1
                cbase = (lv & 1) << 6
                for dd in range(dim // _LANE):
                    vals = []
                    for r in range(_LANE):
                        dv = dd * _LANE + ((lanes + r) & (_LANE - 1))
                        v = plsc.load_gather(s_in.at[p], [dv, lv])
                        vals.append((dv, v))
                    for dv, v in vals:
                        plsc.store_scatter(
                            s_out.at[p], [vpv, cbase | dv], v
                        )
                return c

            lax.fori_loop(0, _GROUP // _LANE, m_body, 0)

        def step(q, c):
            pq = lax.rem(q, nph)

            def phase(p):
                wait_in(p)

                @pl.when(q >= nph)
                def _():
                    wait_out(p)

                transpose(p)

                @pl.when(q + nph < nq)
                def _():
                    start_in(p, q + nph)

                start_out(p, q)

            for p in range(nph):
                @pl.when(pq == p)
                def _(p=p):
                    phase(p)

            return c

        for p in range(nph):
            start_in(p, p)
        lax.fori_loop(0, nq, step, 0)
        for p in range(nph):
            wait_out(p)

        if n_tail:
            @pl.when(wid == 0)
            def _():
                pltpu.sync_copy(tail_hbm, tmp)
                pltpu.sync_copy(
                    tmp,
                    out_hbm.at[pl.ds(n_cols * (_GROUP // 2), n_tail // 2)],
                )

    return pl.kernel(
        body,
        out_type=jax.ShapeDtypeStruct((vocab // 2, 2 * dim), jnp.float32),
        mesh=_MESH,
        scratch_types=[
            pltpu.VMEM((3, dim, _GROUP), jnp.float32),
            pltpu.VMEM((3, _GROUP // 2, 2 * dim), jnp.float32),
            pltpu.VMEM((max(n_tail, 2) // 2, 2 * dim), jnp.float32),
        ]
        + [pltpu.SemaphoreType.DMA] * 6,
        compiler_params=pltpu.CompilerParams(
            use_tc_tiling_on_sc=True, needs_layout_passes=False
        ),
    )


def _gather_call(hist, dim, idx_dtype):
    def body(idx_hbm, table_hbm, out_hbm, idx_v, bufs, tbufs, *sems):
        gsems = sems[:_NBUF]
        wsems = sems[_NBUF:]
        wid = lax.axis_index("s") * _NUM_CORES + lax.axis_index("c")
        lanes = lax.iota(jnp.int32, _LANE)
        pltpu.sync_copy(idx_hbm.at[:, pl.ds(wid * _GROUP, _GROUP)], idx_v)

        def start_gather(b, h):
            pltpu.async_copy(table_hbm.at[idx_v.at[h]], bufs.at[b], gsems[b])

        def wait_gather(b):
            pltpu.make_async_copy(
                table_hbm.at[pl.ds(0, _GROUP)], bufs.at[b], gsems[b]
            ).wait()

        def transpose(b, t, h):
            def jj_body(jj, c):
                rows = lanes + jj * _LANE
                for dd in range(dim // _LANE):
                    vals = []
                    for r in range(_LANE):
                        dloc = dd * _LANE + ((lanes + r) & (_LANE - 1))
                        v = plsc.load_gather(bufs.at[b], [rows, dloc])
                        vals.append((dloc, v))
                    for dloc, v in vals:
                        plsc.store_scatter(
                            tbufs.at[t], [dloc >> 3, dloc & 7, rows], v
                        )
                return c

            lax.fori_loop(0, _GROUP // _LANE, jj_body, 0)

        def start_write(t, h):
            pltpu.async_copy(tbufs.at[t], out_hbm.at[h, :, wid], wsems[t])

        def wait_write(t):
            pltpu.make_async_copy(
                tbufs.at[t], out_hbm.at[0, :, 0], wsems[t]
            ).wait()

        for b in range(_NBUF):
            start_gather(b, b)

        def step(o, c):
            for b in range(_NBUF):
                h = o * _NBUF + b
                t = b % _NTBUF
                wait_gather(b)

                @pl.when(h >= _NTBUF)
                def _():
                    wait_write(t)

                transpose(b, t, h)

                @pl.when(h + _NBUF < hist)
                def _():
                    start_gather(b, h + _NBUF)

                start_write(t, h)
            return c

        lax.fori_loop(0, hist // _NBUF, step, 0)
        for t in range(_NTBUF):
            wait_write(t)

    return pl.kernel(
        body,
        out_type=jax.ShapeDtypeStruct(
            (hist, 8, _NUM_WORKERS, dim // 8, _GROUP), jnp.float32
        ),
        mesh=_MESH,
        scratch_types=[
            pltpu.VMEM((hist, _GROUP), idx_dtype),
            pltpu.VMEM((_NBUF, _GROUP, dim), jnp.float32),
            pltpu.VMEM((_NTBUF, 8, dim // 8, _GROUP), jnp.float32),
        ]
        + [pltpu.SemaphoreType.DMA] * (_NBUF + _NTBUF),
        compiler_params=pltpu.CompilerParams(
            use_tc_tiling_on_sc=False, needs_layout_passes=False
        ),
    )


def kernel(inputs, matrix):
    batch, hist = inputs.shape
    vocab, dim = matrix.shape
    assert batch == _GROUP * _NUM_WORKERS
    n_tail = vocab % _GROUP
    tail = matrix[vocab - n_tail :].reshape(max(n_tail, 2) // 2, 2 * dim)
    packed = _transpose_call(vocab, dim)(matrix.T, tail)
    table_rm = packed.reshape(vocab, dim)
    idx_t = inputs.T
    out5 = _gather_call(hist, dim, idx_t.dtype)(idx_t, table_rm)
    return jnp.transpose(out5, (2, 4, 0, 1, 3)).reshape(batch, hist, dim)


# speedup vs baseline: 3.9008x; 1.0020x over previous
import jax
import jax.numpy as jnp
from jax import lax
from jax.experimental import pallas as pl
from jax.experimental.pallas import tpu as pltpu
from jax.experimental.pallas import tpu_sc as plsc

_NUM_CORES = 2
_NUM_SUBCORES = 16
_NUM_WORKERS = _NUM_CORES * _NUM_SUBCORES

_GROUP = 128
_LANE = 16
_NBUF = 4
_NTBUF = 2

def _mesh():
    return plsc.VectorSubcoreMesh(
        core_axis_name="c",
        subcore_axis_name="s",
        num_cores=_NUM_CORES,
        num_subcores=_NUM_SUBCORES,
    )


def _transpose_call(vocab, dim):
    n_cols = vocab // _GROUP
    n_tail = vocab - n_cols * _GROUP
    base, extra = divmod(n_cols, _NUM_WORKERS)

    nph = 3

    def body(mat_hbm, tail_hbm, out_hbm, s_in, s_out, tmp, *sems):
        isems = sems[:nph]
        osems = sems[nph:]
        wid = lax.axis_index("s") * _NUM_CORES + lax.axis_index("c")
        nq = base + jnp.where(wid < extra, 1, 0)
        lanes = lax.iota(jnp.int32, _LANE)

        def start_in(p, q):
            tc = wid + q * _NUM_WORKERS
            pltpu.async_copy(
                mat_hbm.at[:, pl.ds(tc * _GROUP, _GROUP)], s_in.at[p], isems[p]
            )

        def wait_in(p):
            pltpu.make_async_copy(
                mat_hbm.at[:, pl.ds(0, _GROUP)], s_in.at[p], isems[p]
            ).wait()

        def start_out(p, q):
            tc = wid + q * _NUM_WORKERS
            pltpu.async_copy(
                s_out.at[p], out_hbm.at[pl.ds(tc * (_GROUP // 2), _GROUP // 2)],
                osems[p],
            )

        def wait_out(p):
            pltpu.make_async_copy(
                s_out.at[p], out_hbm.at[pl.ds(0, _GROUP // 2)], osems[p]
            ).wait()

        def transpose(p):
            def m_body(m, c):
                lv = lanes + m * _LANE
                vpv = lv >> 1
                cbase = (lv & 1) << 6
                for dd in range(dim // _LANE):
                    vals = []
                    for r in range(_LANE):
                        dv = dd * _LANE + ((lanes + r) & (_LANE - 1))
                        v = plsc.load_gather(s_in.at[p], [dv, lv])
                        vals.append((dv, v))
                    for dv, v in vals:
                        plsc.store_scatter(
                            s_out.at[p], [vpv, cbase | dv], v
                        )
                return c

            lax.fori_loop(0, _GROUP // _LANE, m_body, 0)

        def step(q, c):
            pq = lax.rem(q, nph)

            def phase(p):
                wait_in(p)

                @pl.when(q >= nph)
                def _():
                    wait_out(p)

                transpose(p)

                @pl.when(q + nph < nq)
                def _():
                    start_in(p, q + nph)

                start_out(p, q)

            for p in range(nph):
                @pl.when(pq == p)
                def _(p=p):
                    phase(p)

            return c

        for p in range(nph):
            start_in(p, p)
        lax.fori_loop(0, nq, step, 0)
        for p in range(nph):
            wait_out(p)

        if n_tail:
            @pl.when(wid == 0)
            def _():
                pltpu.sync_copy(tail_hbm, tmp)
                pltpu.sync_copy(
                    tmp,
                    out_hbm.at[pl.ds(n_cols * (_GROUP // 2), n_tail // 2)],
                )

    return pl.kernel(
        body,
        out_type=jax.ShapeDtypeStruct((vocab // 2, 2 * dim), jnp.float32),
        mesh=_mesh(),
        scratch_types=[
            pltpu.VMEM((3, dim, _GROUP), jnp.float32),
            pltpu.VMEM((3, _GROUP // 2, 2 * dim), jnp.float32),
            pltpu.VMEM((max(n_tail, 2) // 2, 2 * dim), jnp.float32),
        ]
        + [pltpu.SemaphoreType.DMA] * 6,
        compiler_params=pltpu.CompilerParams(
            use_tc_tiling_on_sc=True, needs_layout_passes=False
        ),
    )


def _gather_call(hist, dim, idx_dtype):
    def body(idx_hbm, table_hbm, out_hbm, idx_v, bufs, tbufs, *sems):
        gsems = sems[:_NBUF]
        wsems = sems[_NBUF:]
        wid = lax.axis_index("s") * _NUM_CORES + lax.axis_index("c")
        lanes = lax.iota(jnp.int32, _LANE)
        pltpu.sync_copy(idx_hbm.at[:, pl.ds(wid * _GROUP, _GROUP)], idx_v)

        def start_gather(b, h):
            pltpu.async_copy(table_hbm.at[idx_v.at[h]], bufs.at[b], gsems[b])

        def wait_gather(b):
            pltpu.make_async_copy(
                table_hbm.at[pl.ds(0, _GROUP)], bufs.at[b], gsems[b]
            ).wait()

        def transpose(b, t, h):
            def jj_body(jj, c):
                rows = lanes + jj * _LANE
                for dd in range(dim // _LANE):
                    vals = []
                    for r in range(_LANE):
                        dloc = dd * _LANE + ((lanes + r) & (_LANE - 1))
                        v = plsc.load_gather(bufs.at[b], [rows, dloc])
                        vals.append((dloc, v))
                    for dloc, v in vals:
                        plsc.store_scatter(
                            tbufs.at[t], [dloc >> 3, dloc & 7, rows], v
                        )
                return c

            lax.fori_loop(0, _GROUP // _LANE, jj_body, 0)

        def start_write(t, h):
            pltpu.async_copy(tbufs.at[t], out_hbm.at[h, :, wid], wsems[t])

        def wait_write(t):
            pltpu.make_async_copy(
                tbufs.at[t], out_hbm.at[0, :, 0], wsems[t]
            ).wait()

        for b in range(_NBUF):
            start_gather(b, b)

        def step(o, c):
            for b in range(_NBUF):
                h = o * _NBUF + b
                t = b % _NTBUF
                wait_gather(b)

                @pl.when(h >= _NTBUF)
                def _():
                    wait_write(t)

                transpose(b, t, h)

                @pl.when(h + _NBUF < hist)
                def _():
                    start_gather(b, h + _NBUF)

                start_write(t, h)
            return c

        lax.fori_loop(0, hist // _NBUF, step, 0)
        for t in range(_NTBUF):
            wait_write(t)

    return pl.kernel(
        body,
        out_type=jax.ShapeDtypeStruct(
            (hist, 8, _NUM_WORKERS, dim // 8, _GROUP), jnp.float32
        ),
        mesh=_mesh(),
        scratch_types=[
            pltpu.VMEM((hist, _GROUP), idx_dtype),
            pltpu.VMEM((_NBUF, _GROUP, dim), jnp.float32),
            pltpu.VMEM((_NTBUF, 8, dim // 8, _GROUP), jnp.float32),
        ]
        + [pltpu.SemaphoreType.DMA] * (_NBUF + _NTBUF),
        compiler_params=pltpu.CompilerParams(
            use_tc_tiling_on_sc=False, needs_layout_passes=False
        ),
    )


def kernel(inputs, matrix):
    batch, hist = inputs.shape
    vocab, dim = matrix.shape
    assert batch == _GROUP * _NUM_WORKERS
    n_tail = vocab % _GROUP
    tail = matrix[vocab - n_tail :].reshape(max(n_tail, 2) // 2, 2 * dim)
    packed = _transpose_call(vocab, dim)(matrix.T, tail)
    table_rm = packed.reshape(vocab, dim)
    idx_t = inputs.T
    out5 = _gather_call(hist, dim, idx_t.dtype)(idx_t, table_rm)
    return jnp.transpose(out5, (2, 4, 0, 1, 3)).reshape(batch, hist, dim)
